# Initial kernel scaffold; baseline (speedup 1.0000x reference)
import functools

import jax
import jax.numpy as jnp
from jax import lax
from jax.experimental import pallas as pl
from jax.experimental.pallas import tpu as pltpu
from jax.experimental.pallas import tpu_sc as plsc

N = 10000
E = 320000
D = 128
H = 8
U = 16
HP = 16
CHUNK = 128
NCHUNKS = E // CHUNK
NC = 2
NS = 16
NW = NC * NS
FULL_ROUNDS = NCHUNKS // NW
REM = NCHUNKS - FULL_ROUNDS * NW
RPS = N // NS


def _tc_project(x, w, ba):
    rb = 1000

    def body(x_ref, w_ref, ba_ref, xn_ref, xb_ref):
        xn = jnp.dot(x_ref[...], w_ref[...], preferred_element_type=jnp.float32)
        xn_ref[...] = xn
        xb_ref[...] = xn + ba_ref[...]

    return pl.pallas_call(
        body,
        grid=(N // rb,),
        in_specs=[
            pl.BlockSpec((rb, D), lambda i: (i, 0)),
            pl.BlockSpec((D, H * U), lambda i: (0, 0)),
            pl.BlockSpec((1, H * U), lambda i: (0, 0)),
        ],
        out_specs=[
            pl.BlockSpec((rb, H * U), lambda i: (i, 0)),
            pl.BlockSpec((rb, H * U), lambda i: (i, 0)),
        ],
        out_shape=[
            jax.ShapeDtypeStruct((N, H * U), jnp.float32),
            jax.ShapeDtypeStruct((N, H * U), jnp.float32),
        ],
    )(x, w, ba)


def _sc_scores(xb, tgt, src, ka1, zeros16):
    mesh = plsc.VectorSubcoreMesh(core_axis_name="c", subcore_axis_name="s")

    @functools.partial(
        pl.kernel,
        out_type=(
            jax.ShapeDtypeStruct((E, HP), jnp.float32),
            jax.ShapeDtypeStruct((NC, N, HP), jnp.float32),
        ),
        mesh=mesh,
        scratch_types=[
            pltpu.VMEM_SHARED((N, HP), jnp.float32),
            pltpu.VMEM((CHUNK,), jnp.int32),
            pltpu.VMEM((CHUNK,), jnp.int32),
            pltpu.VMEM((CHUNK, D), jnp.float32),
            pltpu.VMEM((CHUNK, D), jnp.float32),
            pltpu.VMEM((CHUNK, HP), jnp.float32),
            pltpu.VMEM((D,), jnp.float32),
        ],
    )
    def k(xb_hbm, tgt_hbm, src_hbm, ka1_hbm, z_hbm, p_hbm, ssum_hbm,
          acc, tgt_v, src_v, ft_v, fs_v, p_v, ka1_v):
        c = lax.axis_index("c")
        s = lax.axis_index("s")
        w = s * NC + c
        pltpu.sync_copy(ka1_hbm, ka1_v)
        ka = [ka1_v[pl.ds(h * U, U)] for h in range(H)]
        pltpu.sync_copy(
            z_hbm.at[pl.ds(s * RPS, RPS)], acc.at[pl.ds(s * RPS, RPS)])
        plsc.subcore_barrier()

        def do_chunk(cid):
            base = cid * CHUNK
            pltpu.sync_copy(tgt_hbm.at[pl.ds(base, CHUNK)], tgt_v)
            pltpu.sync_copy(src_hbm.at[pl.ds(base, CHUNK)], src_v)
            pltpu.sync_copy(xb_hbm.at[tgt_v], ft_v)
            pltpu.sync_copy(xb_hbm.at[src_v], fs_v)

            @pl.loop(0, CHUNK)
            def _(i):
                for h in range(H):
                    z = ft_v[i, pl.ds(h * U, U)] + fs_v[i, pl.ds(h * U, U)]
                    t = jnp.maximum(z, 0.2 * z) * ka[h]
                    p_v[i, h] = jnp.sum(t)
                row = p_v[i, :]
                lane = lax.iota(jnp.int32, HP)
                p_v[i, :] = jnp.where(lane < H, jnp.exp(row), 0.0)

            pltpu.sync_copy(p_v, p_hbm.at[pl.ds(base, CHUNK)])
            pltpu.sync_copy(p_v, acc.at[tgt_v], add=True)

        @pl.loop(0, FULL_ROUNDS)
        def _(r):
            do_chunk(w + r * NW)

        @pl.when(w < REM)
        def _():
            do_chunk(FULL_ROUNDS * NW + w)

        plsc.subcore_barrier()
        pltpu.sync_copy(
            acc.at[pl.ds(s * RPS, RPS)],
            ssum_hbm.at[c, pl.ds(s * RPS, RPS)])

    return k(xb, tgt, src, ka1, zeros16)


def _tc_rinv(ssum_p):
    rows = (N * HP) // 128
    sp = ssum_p.reshape(NC, rows, 128)

    def body(s_ref, o_ref):
        o_ref[...] = 1.0 / (s_ref[0] + s_ref[1] + 1e-7)

    r = pl.pallas_call(
        body,
        out_shape=jax.ShapeDtypeStruct((rows, 128), jnp.float32),
    )(sp)
    return r.reshape(N, HP)


def _sc_aggregate(xn, tgt, src, p, rinv, zeros128):
    mesh = plsc.VectorSubcoreMesh(core_axis_name="c", subcore_axis_name="s")

    @functools.partial(
        pl.kernel,
        out_type=jax.ShapeDtypeStruct((NC, N, D), jnp.float32),
        mesh=mesh,
        scratch_types=[
            pltpu.VMEM_SHARED((N, D), jnp.float32),
            pltpu.VMEM((CHUNK,), jnp.int32),
            pltpu.VMEM((CHUNK,), jnp.int32),
            pltpu.VMEM((CHUNK, D), jnp.float32),
            pltpu.VMEM((CHUNK, D), jnp.float32),
            pltpu.VMEM((CHUNK, HP), jnp.float32),
            pltpu.VMEM((CHUNK, HP), jnp.float32),
        ],
    )
    def k(xn_hbm, tgt_hbm, src_hbm, p_hbm, rinv_hbm, z_hbm, out_hbm,
          acc, tgt_v, src_v, xs_v, msg_v, p_v, rinv_v):
        c = lax.axis_index("c")
        s = lax.axis_index("s")
        w = s * NC + c
        pltpu.sync_copy(
            z_hbm.at[pl.ds(s * RPS, RPS)], acc.at[pl.ds(s * RPS, RPS)])
        plsc.subcore_barrier()

        def do_chunk(cid):
            base = cid * CHUNK
            pltpu.sync_copy(tgt_hbm.at[pl.ds(base, CHUNK)], tgt_v)
            pltpu.sync_copy(src_hbm.at[pl.ds(base, CHUNK)], src_v)
            pltpu.sync_copy(p_hbm.at[pl.ds(base, CHUNK)], p_v)
            pltpu.sync_copy(rinv_hbm.at[tgt_v], rinv_v)
            pltpu.sync_copy(xn_hbm.at[src_v], xs_v)

            @pl.loop(0, CHUNK)
            def _(i):
                for h in range(H):
                    wh = p_v[i, h] * rinv_v[i, h]
                    msg_v[i, pl.ds(h * U, U)] = xs_v[i, pl.ds(h * U, U)] * wh

            pltpu.sync_copy(msg_v, acc.at[tgt_v], add=True)

        @pl.loop(0, FULL_ROUNDS)
        def _(r):
            do_chunk(w + r * NW)

        @pl.when(w < REM)
        def _():
            do_chunk(FULL_ROUNDS * NW + w)

        plsc.subcore_barrier()
        pltpu.sync_copy(
            acc.at[pl.ds(s * RPS, RPS)],
            out_hbm.at[c, pl.ds(s * RPS, RPS)])

    return k(xn, tgt, src, p, rinv, zeros128)


def _tc_finish(acc, bias):
    rb = 1000

    def body(a_ref, b_ref, o_ref):
        o_ref[...] = jax.nn.gelu(a_ref[0] + a_ref[1] + b_ref[...])

    return pl.pallas_call(
        body,
        grid=(N // rb,),
        in_specs=[
            pl.BlockSpec((NC, rb, D), lambda i: (0, i, 0)),
            pl.BlockSpec((1, D), lambda i: (0, 0)),
        ],
        out_specs=pl.BlockSpec((rb, D), lambda i: (i, 0)),
        out_shape=jax.ShapeDtypeStruct((N, D), jnp.float32),
    )(acc, bias.reshape(1, D))


def kernel(x, edges, kernel, kernel_attention1, bias_attention, bias):
    w = kernel.reshape(D, H * U)
    ka1 = kernel_attention1.reshape(H * U)
    ba = bias_attention.reshape(1, H * U)
    tgt = edges[:, 1]
    src = edges[:, 0]
    zeros16 = jnp.zeros((N, HP), jnp.float32)
    zeros128 = jnp.zeros((N, D), jnp.float32)

    xn, xbias = _tc_project(x, w, ba)
    p, ssum_p = _sc_scores(xbias, tgt, src, ka1, zeros16)
    rinv = _tc_rinv(ssum_p)
    acc = _sc_aggregate(xn, tgt, src, p, rinv, zeros128)
    return _tc_finish(acc, bias)


# speedup vs baseline: 30.0014x; 30.0014x over previous
import dataclasses
import functools

import jax
import jax.numpy as jnp
from jax import lax
from jax.experimental import pallas as pl
from jax.experimental.pallas import tpu as pltpu
from jax.experimental.pallas import tpu_sc as plsc

N = 10000
E = 320000
D = 128
H = 8
U = 16
HP = 16
CHUNK = 128
NCHUNKS = E // CHUNK
NC = 2
NS = 16
NW = NC * NS
FULL_ROUNDS = NCHUNKS // NW
REM = NCHUNKS - FULL_ROUNDS * NW
_SC_CP = pltpu.CompilerParams()
if "needs_layout_passes" in pltpu.CompilerParams.__dataclass_fields__:
    _SC_CP = dataclasses.replace(_SC_CP, needs_layout_passes=False)
if "use_tc_tiling_on_sc" in pltpu.CompilerParams.__dataclass_fields__:
    _SC_CP = dataclasses.replace(_SC_CP, use_tc_tiling_on_sc=False)

NP = 10240
RPS = NP // NS


def _tc_project(x, w, ba):
    rb = 1000

    def body(x_ref, w_ref, ba_ref, xn_ref, xb_ref):
        xn = jnp.dot(x_ref[...], w_ref[...], preferred_element_type=jnp.float32)
        xn_ref[...] = xn
        xb_ref[...] = xn + ba_ref[...]

    return pl.pallas_call(
        body,
        grid=(N // rb,),
        in_specs=[
            pl.BlockSpec((rb, D), lambda i: (i, 0)),
            pl.BlockSpec((D, H * U), lambda i: (0, 0)),
            pl.BlockSpec((1, H * U), lambda i: (0, 0)),
        ],
        out_specs=[
            pl.BlockSpec((rb, H * U), lambda i: (i, 0)),
            pl.BlockSpec((rb, H * U), lambda i: (i, 0)),
        ],
        out_shape=[
            jax.ShapeDtypeStruct((N, H * U), jnp.float32),
            jax.ShapeDtypeStruct((N, H * U), jnp.float32),
        ],
    )(x, w, ba)


def _sc_scores(xb, tgt, src, ka1, zeros16):
    mesh = plsc.VectorSubcoreMesh(core_axis_name="c", subcore_axis_name="s")

    @functools.partial(
        pl.kernel,
        out_type=(
            jax.ShapeDtypeStruct((E, HP), jnp.float32),
            jax.ShapeDtypeStruct((NC, NP, HP), jnp.float32),
        ),
        mesh=mesh,
        compiler_params=_SC_CP,
        scratch_types=[
            pltpu.VMEM_SHARED((NP, HP), jnp.float32),
            pltpu.VMEM((CHUNK,), jnp.int32),
            pltpu.VMEM((CHUNK,), jnp.int32),
            pltpu.VMEM((CHUNK, D), jnp.float32),
            pltpu.VMEM((CHUNK, D), jnp.float32),
            pltpu.VMEM((CHUNK, HP), jnp.float32),
            pltpu.VMEM((D,), jnp.float32),
        ],
    )
    def k(xb_hbm, tgt_hbm, src_hbm, ka1_hbm, z_hbm, p_hbm, ssum_hbm,
          acc, tgt_v, src_v, ft_v, fs_v, p_v, ka1_v):
        c = lax.axis_index("c")
        s = lax.axis_index("s")
        w = s * NC + c
        pltpu.sync_copy(ka1_hbm, ka1_v)
        ka = [ka1_v[pl.ds(h * U, U)] for h in range(H)]
        pltpu.sync_copy(
            z_hbm.at[pl.ds(s * RPS, RPS)], acc.at[pl.ds(s * RPS, RPS)])
        plsc.subcore_barrier()

        def do_chunk(cid):
            base = cid * CHUNK
            pltpu.sync_copy(tgt_hbm.at[pl.ds(base, CHUNK)], tgt_v)
            pltpu.sync_copy(src_hbm.at[pl.ds(base, CHUNK)], src_v)
            pltpu.sync_copy(xb_hbm.at[tgt_v], ft_v)
            pltpu.sync_copy(xb_hbm.at[src_v], fs_v)

            @pl.loop(0, CHUNK)
            def _(i):
                lane = lax.iota(jnp.int32, HP)
                row = jnp.zeros((HP,), jnp.float32)
                for h in range(H):
                    z = ft_v[i, pl.ds(h * U, U)] + fs_v[i, pl.ds(h * U, U)]
                    t = jnp.maximum(z, 0.2 * z) * ka[h]
                    row = jnp.where(lane == h, jnp.sum(t), row)
                p_v[i, :] = jnp.where(lane < H, jnp.exp(row), 0.0)

            pltpu.sync_copy(p_v, p_hbm.at[pl.ds(base, CHUNK)])
            pltpu.sync_copy(p_v, acc.at[tgt_v], add=True)

        @pl.loop(0, FULL_ROUNDS)
        def _(r):
            do_chunk(w + r * NW)

        @pl.when(w < REM)
        def _():
            do_chunk(FULL_ROUNDS * NW + w)

        plsc.subcore_barrier()
        pltpu.sync_copy(
            acc.at[pl.ds(s * RPS, RPS)],
            ssum_hbm.at[c, pl.ds(s * RPS, RPS)])

    return k(xb, tgt, src, ka1, zeros16)


def _tc_rinv(ssum_p):
    rows = (NP * HP) // 128
    sp = ssum_p.reshape(NC, rows, 128)

    def body(s_ref, o_ref):
        o_ref[...] = 1.0 / (s_ref[0] + s_ref[1] + 1e-7)

    r = pl.pallas_call(
        body,
        out_shape=jax.ShapeDtypeStruct((rows, 128), jnp.float32),
    )(sp)
    return r.reshape(NP, HP)


def _sc_aggregate(xn, tgt, src, p, rinv, zeros128):
    mesh = plsc.VectorSubcoreMesh(core_axis_name="c", subcore_axis_name="s")

    @functools.partial(
        pl.kernel,
        out_type=jax.ShapeDtypeStruct((NC, NP, D), jnp.float32),
        mesh=mesh,
        compiler_params=_SC_CP,
        scratch_types=[
            pltpu.VMEM_SHARED((NP, D), jnp.float32),
            pltpu.VMEM((CHUNK,), jnp.int32),
            pltpu.VMEM((CHUNK,), jnp.int32),
            pltpu.VMEM((CHUNK, D), jnp.float32),
            pltpu.VMEM((CHUNK, D), jnp.float32),
            pltpu.VMEM((CHUNK, HP), jnp.float32),
            pltpu.VMEM((CHUNK, HP), jnp.float32),
        ],
    )
    def k(xn_hbm, tgt_hbm, src_hbm, p_hbm, rinv_hbm, z_hbm, out_hbm,
          acc, tgt_v, src_v, xs_v, msg_v, p_v, rinv_v):
        c = lax.axis_index("c")
        s = lax.axis_index("s")
        w = s * NC + c
        pltpu.sync_copy(
            z_hbm.at[pl.ds(s * RPS, RPS)], acc.at[pl.ds(s * RPS, RPS)])
        plsc.subcore_barrier()

        def do_chunk(cid):
            base = cid * CHUNK
            pltpu.sync_copy(tgt_hbm.at[pl.ds(base, CHUNK)], tgt_v)
            pltpu.sync_copy(src_hbm.at[pl.ds(base, CHUNK)], src_v)
            pltpu.sync_copy(p_hbm.at[pl.ds(base, CHUNK)], p_v)
            pltpu.sync_copy(rinv_hbm.at[tgt_v], rinv_v)
            pltpu.sync_copy(xn_hbm.at[src_v], xs_v)

            @pl.loop(0, CHUNK)
            def _(i):
                w16 = p_v[i, :] * rinv_v[i, :]
                for h in range(H):
                    msg_v[i, pl.ds(h * U, U)] = xs_v[i, pl.ds(h * U, U)] * w16[h]

            pltpu.sync_copy(msg_v, acc.at[tgt_v], add=True)

        @pl.loop(0, FULL_ROUNDS)
        def _(r):
            do_chunk(w + r * NW)

        @pl.when(w < REM)
        def _():
            do_chunk(FULL_ROUNDS * NW + w)

        plsc.subcore_barrier()
        pltpu.sync_copy(
            acc.at[pl.ds(s * RPS, RPS)],
            out_hbm.at[c, pl.ds(s * RPS, RPS)])

    return k(xn, tgt, src, p, rinv, zeros128)


def _tc_finish(acc, bias):
    rb = 1000

    def body(a_ref, b_ref, o_ref):
        o_ref[...] = jax.nn.gelu(a_ref[0] + a_ref[1] + b_ref[...])

    return pl.pallas_call(
        body,
        grid=(N // rb,),
        in_specs=[
            pl.BlockSpec((NC, rb, D), lambda i: (0, i, 0)),
            pl.BlockSpec((1, D), lambda i: (0, 0)),
        ],
        out_specs=pl.BlockSpec((rb, D), lambda i: (i, 0)),
        out_shape=jax.ShapeDtypeStruct((N, D), jnp.float32),
    )(acc, bias.reshape(1, D))


def kernel(x, edges, kernel, kernel_attention1, bias_attention, bias):
    w = kernel.reshape(D, H * U)
    ka1 = kernel_attention1.reshape(H * U)
    ba = bias_attention.reshape(1, H * U)
    tgt = edges[:, 1]
    src = edges[:, 0]
    zeros16 = jnp.zeros((NP, HP), jnp.float32)
    zeros128 = jnp.zeros((NP, D), jnp.float32)

    xn, xbias = _tc_project(x, w, ba)
    p, ssum_p = _sc_scores(xbias, tgt, src, ka1, zeros16)
    rinv = _tc_rinv(ssum_p)
    acc = _sc_aggregate(xn, tgt, src, p, rinv, zeros128)
    return _tc_finish(acc, bias)


# speedup vs baseline: 32.6056x; 1.0868x over previous
import dataclasses
import functools

import jax
import jax.numpy as jnp
from jax import lax
from jax.experimental import pallas as pl
from jax.experimental.pallas import tpu as pltpu
from jax.experimental.pallas import tpu_sc as plsc

N = 10000
E = 320000
D = 128
H = 8
U = 16
HP = 16
CHUNK = 128
NC = 2
NS = 16
NW = NC * NS
CPT = 80
EPT = CPT * CHUNK
E_PAD = NW * EPT
NCH = E_PAD // CHUNK
NP = 10240
RPS = NP // NS

_SC_CP = pltpu.CompilerParams()
if "needs_layout_passes" in pltpu.CompilerParams.__dataclass_fields__:
    _SC_CP = dataclasses.replace(_SC_CP, needs_layout_passes=False)
if "use_tc_tiling_on_sc" in pltpu.CompilerParams.__dataclass_fields__:
    _SC_CP = dataclasses.replace(_SC_CP, use_tc_tiling_on_sc=False)


def _tc_project(xp, w, ba):
    rb = 1024

    def body(x_ref, w_ref, ba_ref, xn_ref, xb_ref):
        xn = jnp.dot(x_ref[...], w_ref[...], preferred_element_type=jnp.float32)
        xn_ref[...] = xn
        xb_ref[...] = xn + ba_ref[...]

    return pl.pallas_call(
        body,
        grid=(NP // rb,),
        in_specs=[
            pl.BlockSpec((rb, D), lambda i: (i, 0)),
            pl.BlockSpec((D, H * U), lambda i: (0, 0)),
            pl.BlockSpec((1, H * U), lambda i: (0, 0)),
        ],
        out_specs=[
            pl.BlockSpec((rb, H * U), lambda i: (i, 0)),
            pl.BlockSpec((rb, H * U), lambda i: (i, 0)),
        ],
        out_shape=[
            jax.ShapeDtypeStruct((NP, H * U), jnp.float32),
            jax.ShapeDtypeStruct((NP, H * U), jnp.float32),
        ],
    )(xp, w, ba)


def _sc_scores(xb, tgt2d, src2d, ka1, zeros16):
    mesh = plsc.VectorSubcoreMesh(core_axis_name="c", subcore_axis_name="s")

    @functools.partial(
        pl.kernel,
        out_type=(
            jax.ShapeDtypeStruct((E_PAD, HP), jnp.float32),
            jax.ShapeDtypeStruct((NC, NP, HP), jnp.float32),
        ),
        mesh=mesh,
        compiler_params=_SC_CP,
        scratch_types=[
            pltpu.VMEM_SHARED((NP, HP), jnp.float32),
            pltpu.VMEM((CPT, CHUNK), jnp.int32),
            pltpu.VMEM((CPT, CHUNK), jnp.int32),
            pltpu.VMEM((CHUNK, D), jnp.float32),
            pltpu.VMEM((CHUNK, D), jnp.float32),
            pltpu.VMEM((CHUNK, D), jnp.float32),
            pltpu.VMEM((CHUNK, D), jnp.float32),
            pltpu.VMEM((CHUNK, HP), jnp.float32),
            pltpu.VMEM((D,), jnp.float32),
            pltpu.SemaphoreType.DMA,
            pltpu.SemaphoreType.DMA,
            pltpu.SemaphoreType.DMA,
            pltpu.SemaphoreType.DMA,
        ],
    )
    def k(xb_hbm, tgt_hbm, src_hbm, ka1_hbm, z_hbm, p_hbm, ssum_hbm,
          acc, tgtv, srcv, ft0, ft1, fs0, fs1, p_v, ka1_v,
          sf0, sf1, ss0, ss1):
        c = lax.axis_index("c")
        s = lax.axis_index("s")
        w = s * NC + c
        ft = (ft0, ft1)
        fs = (fs0, fs1)
        sf = (sf0, sf1)
        ss = (ss0, ss1)
        pltpu.sync_copy(ka1_hbm, ka1_v)
        ka = [ka1_v[pl.ds(h * U, U)] for h in range(H)]
        pltpu.sync_copy(tgt_hbm.at[pl.ds(w * CPT, CPT)], tgtv)
        pltpu.sync_copy(src_hbm.at[pl.ds(w * CPT, CPT)], srcv)
        pltpu.sync_copy(
            z_hbm.at[pl.ds(s * RPS, RPS)], acc.at[pl.ds(s * RPS, RPS)])
        plsc.subcore_barrier()

        def descs(r, b):
            return (pltpu.make_async_copy(xb_hbm.at[tgtv.at[r]], ft[b], sf[b]),
                    pltpu.make_async_copy(xb_hbm.at[srcv.at[r]], fs[b], ss[b]))

        def issue(r, b):
            d1, d2 = descs(r, b)
            d1.start()
            d2.start()

        def compute_tail(r, b):
            d1, d2 = descs(r, b)
            d1.wait()
            d2.wait()

            @pl.loop(0, CHUNK)
            def _(i):
                lane = lax.iota(jnp.int32, HP)
                row = jnp.zeros((HP,), jnp.float32)
                for h in range(H):
                    z = ft[b][i, pl.ds(h * U, U)] + fs[b][i, pl.ds(h * U, U)]
                    t = jnp.maximum(z, 0.2 * z) * ka[h]
                    row = jnp.where(lane == h, jnp.sum(t), row)
                p_v[i, :] = jnp.where(lane < H, jnp.exp(row), 0.0)

            base = (w * CPT + r) * CHUNK
            pltpu.sync_copy(p_v, p_hbm.at[pl.ds(base, CHUNK)])
            pltpu.sync_copy(p_v, acc.at[tgtv.at[r]], add=True)

        issue(0, 0)

        @pl.loop(0, CPT, step=2)
        def _(rr):
            issue(rr + 1, 1)
            compute_tail(rr, 0)

            @pl.when(rr + 2 < CPT)
            def _():
                issue(rr + 2, 0)

            compute_tail(rr + 1, 1)

        plsc.subcore_barrier()
        pltpu.sync_copy(
            acc.at[pl.ds(s * RPS, RPS)],
            ssum_hbm.at[c, pl.ds(s * RPS, RPS)])

    return k(xb, tgt2d, src2d, ka1, zeros16)


def _tc_rinvx(ssum_p):
    rb = 1024

    def body(s_ref, o_ref):
        r = 1.0 / (s_ref[0, :, :H] + s_ref[1, :, :H] + 1e-7)
        o_ref[...] = jnp.repeat(r, U, axis=1)

    return pl.pallas_call(
        body,
        grid=(NP // rb,),
        in_specs=[pl.BlockSpec((NC, rb, HP), lambda i: (0, i, 0))],
        out_specs=pl.BlockSpec((rb, D), lambda i: (i, 0)),
        out_shape=jax.ShapeDtypeStruct((NP, D), jnp.float32),
    )(ssum_p)


def _sc_aggregate(xn, tgt2d, src2d, p, zeros128):
    mesh = plsc.VectorSubcoreMesh(core_axis_name="c", subcore_axis_name="s")
    CPH = CPT // 2

    @functools.partial(
        pl.kernel,
        out_type=jax.ShapeDtypeStruct((NC, NP, D), jnp.float32),
        mesh=mesh,
        compiler_params=_SC_CP,
        scratch_types=[
            pltpu.VMEM_SHARED((NP, D), jnp.float32),
            pltpu.VMEM((CPH, CHUNK), jnp.int32),
            pltpu.VMEM((CPH, CHUNK), jnp.int32),
            pltpu.VMEM((CHUNK, D), jnp.float32),
            pltpu.VMEM((CHUNK, D), jnp.float32),
            pltpu.VMEM((CHUNK, HP), jnp.float32),
            pltpu.VMEM((CHUNK, HP), jnp.float32),
            pltpu.SemaphoreType.DMA,
            pltpu.SemaphoreType.DMA,
            pltpu.SemaphoreType.DMA,
            pltpu.SemaphoreType.DMA,
        ],
    )
    def k(xn_hbm, tgt_hbm, src_hbm, p_hbm, z_hbm, out_hbm,
          acc, tgtv, srcv, xs0, xs1, p0, p1,
          sx0, sx1, sp0, sp1):
        c = lax.axis_index("c")
        s = lax.axis_index("s")
        w = s * NC + c
        xs = (xs0, xs1)
        pv = (p0, p1)
        sx = (sx0, sx1)
        sp = (sp0, sp1)
        pltpu.sync_copy(
            z_hbm.at[pl.ds(s * RPS, RPS)], acc.at[pl.ds(s * RPS, RPS)])
        plsc.subcore_barrier()

        for ph in range(2):
            pltpu.sync_copy(
                tgt_hbm.at[pl.ds(w * CPT + ph * CPH, CPH)], tgtv)
            pltpu.sync_copy(
                src_hbm.at[pl.ds(w * CPT + ph * CPH, CPH)], srcv)

            def descs(r, b):
                base = (w * CPT + ph * CPH + r) * CHUNK
                return (
                    pltpu.make_async_copy(xn_hbm.at[srcv.at[r]], xs[b], sx[b]),
                    pltpu.make_async_copy(
                        p_hbm.at[pl.ds(base, CHUNK)], pv[b], sp[b]))

            def issue(r, b):
                d1, d2 = descs(r, b)
                d1.start()
                d2.start()

            def compute_tail(r, b):
                d1, d2 = descs(r, b)
                d1.wait()
                d2.wait()

                @pl.loop(0, CHUNK)
                def _(i):
                    w16 = pv[b][i, :]
                    for h in range(H):
                        xs[b][i, pl.ds(h * U, U)] = (
                            xs[b][i, pl.ds(h * U, U)] * w16[h])

                pltpu.sync_copy(xs[b], acc.at[tgtv.at[r]], add=True)

            issue(0, 0)

            @pl.loop(0, CPH, step=2)
            def _(rr):
                issue(rr + 1, 1)
                compute_tail(rr, 0)

                @pl.when(rr + 2 < CPH)
                def _():
                    issue(rr + 2, 0)

                compute_tail(rr + 1, 1)

        plsc.subcore_barrier()
        pltpu.sync_copy(
            acc.at[pl.ds(s * RPS, RPS)],
            out_hbm.at[c, pl.ds(s * RPS, RPS)])

    return k(xn, tgt2d, src2d, p, zeros128)


def _tc_finish(acc, rinvx, bias):
    rb = 1000

    def body(a_ref, r_ref, b_ref, o_ref):
        o_ref[...] = jax.nn.gelu(
            (a_ref[0] + a_ref[1]) * r_ref[...] + b_ref[...])

    return pl.pallas_call(
        body,
        grid=(N // rb,),
        in_specs=[
            pl.BlockSpec((NC, rb, D), lambda i: (0, i, 0)),
            pl.BlockSpec((rb, D), lambda i: (i, 0)),
            pl.BlockSpec((1, D), lambda i: (0, 0)),
        ],
        out_specs=pl.BlockSpec((rb, D), lambda i: (i, 0)),
        out_shape=jax.ShapeDtypeStruct((N, D), jnp.float32),
    )(acc, rinvx, bias.reshape(1, D))


def kernel(x, edges, kernel, kernel_attention1, bias_attention, bias):
    w = kernel.reshape(D, H * U)
    ka1 = kernel_attention1.reshape(H * U)
    ba = bias_attention.reshape(1, H * U)
    pad = jnp.full((E_PAD - E,), N, jnp.int32)
    tgt2d = jnp.concatenate([edges[:, 1], pad]).reshape(NCH, CHUNK)
    src2d = jnp.concatenate([edges[:, 0], pad]).reshape(NCH, CHUNK)
    xp = jnp.pad(x, ((0, NP - N), (0, 0)))
    zeros16 = jnp.zeros((NP, HP), jnp.float32)
    zeros128 = jnp.zeros((NP, D), jnp.float32)

    xn, xbias = _tc_project(xp, w, ba)
    p, ssum_p = _sc_scores(xbias, tgt2d, src2d, ka1, zeros16)
    rinvx = _tc_rinvx(ssum_p)
    acc = _sc_aggregate(xn, tgt2d, src2d, p, zeros128)
    return _tc_finish(acc, rinvx, bias)


# speedup vs baseline: 124.1554x; 3.8078x over previous
import dataclasses
import functools

import jax
import jax.numpy as jnp
from jax import lax
from jax.experimental import pallas as pl
from jax.experimental.pallas import tpu as pltpu
from jax.experimental.pallas import tpu_sc as plsc

N = 10000
E = 320000
D = 128
H = 8
U = 16
HP = 16
CHUNK = 64
NC = 2
NS = 16
NW = NC * NS
CPT = 160
EPT = CPT * CHUNK
E_PAD = NW * EPT
NCH = E_PAD // CHUNK
CPH = 32
NPH = CPT // CPH
NP = 10240
RPS = NP // NS

_SC_CP = pltpu.CompilerParams()
if "needs_layout_passes" in pltpu.CompilerParams.__dataclass_fields__:
    _SC_CP = dataclasses.replace(_SC_CP, needs_layout_passes=False)
if "use_tc_tiling_on_sc" in pltpu.CompilerParams.__dataclass_fields__:
    _SC_CP = dataclasses.replace(_SC_CP, use_tc_tiling_on_sc=False)


def _tc_project(xp, w):
    rb = 1024

    def body(x_ref, w_ref, xn_ref):
        xn_ref[...] = jnp.dot(
            x_ref[...], w_ref[...], preferred_element_type=jnp.float32)

    return pl.pallas_call(
        body,
        grid=(NP // rb,),
        in_specs=[
            pl.BlockSpec((rb, D), lambda i: (i, 0)),
            pl.BlockSpec((D, H * U), lambda i: (0, 0)),
        ],
        out_specs=pl.BlockSpec((rb, H * U), lambda i: (i, 0)),
        out_shape=jax.ShapeDtypeStruct((NP, H * U), jnp.float32),
    )(xp, w)


def _sc_fused(xn, tgt2d, src2d, ka1, tba, zeros16, zeros128):
    mesh = plsc.VectorSubcoreMesh(core_axis_name="c", subcore_axis_name="s")

    @functools.partial(
        pl.kernel,
        out_type=(
            jax.ShapeDtypeStruct((NC, NP, HP), jnp.float32),
            jax.ShapeDtypeStruct((NC, NP, D), jnp.float32),
        ),
        mesh=mesh,
        compiler_params=_SC_CP,
        scratch_types=[
            pltpu.VMEM_SHARED((NP, HP), jnp.float32),
            pltpu.VMEM_SHARED((NP, D), jnp.float32),
            pltpu.VMEM((CPH, CHUNK), jnp.int32),
            pltpu.VMEM((CPH, CHUNK), jnp.int32),
            pltpu.VMEM((CHUNK, D), jnp.float32),
            pltpu.VMEM((CHUNK, D), jnp.float32),
            pltpu.VMEM((CHUNK, D), jnp.float32),
            pltpu.VMEM((CHUNK, D), jnp.float32),
            pltpu.VMEM((CHUNK, HP), jnp.float32),
            pltpu.VMEM((D,), jnp.float32),
            pltpu.VMEM((D,), jnp.float32),
            pltpu.SemaphoreType.DMA,
            pltpu.SemaphoreType.DMA,
            pltpu.SemaphoreType.DMA,
            pltpu.SemaphoreType.DMA,
        ],
    )
    def k(xn_hbm, tgt_hbm, src_hbm, ka1_hbm, tba_hbm, z16_hbm, z128_hbm,
          ssum_hbm, out_hbm,
          acc16, acc128, tgtv, srcv, ft0, ft1, fs0, fs1, p_v, ka1_v, tba_v,
          sf0, sf1, ss0, ss1):
        c = lax.axis_index("c")
        s = lax.axis_index("s")
        w = s * NC + c
        ft = (ft0, ft1)
        fs = (fs0, fs1)
        sf = (sf0, sf1)
        ss = (ss0, ss1)
        pltpu.sync_copy(ka1_hbm, ka1_v)
        pltpu.sync_copy(tba_hbm, tba_v)
        ka = [ka1_v[pl.ds(h * U, U)] for h in range(H)]
        tb = [tba_v[pl.ds(h * U, U)] for h in range(H)]
        pltpu.sync_copy(
            z16_hbm.at[pl.ds(s * RPS, RPS)], acc16.at[pl.ds(s * RPS, RPS)])
        pltpu.sync_copy(
            z128_hbm.at[pl.ds(s * RPS, RPS)], acc128.at[pl.ds(s * RPS, RPS)])
        plsc.subcore_barrier()

        for ph in range(NPH):
            pltpu.sync_copy(
                tgt_hbm.at[pl.ds(w * CPT + ph * CPH, CPH)], tgtv)
            pltpu.sync_copy(
                src_hbm.at[pl.ds(w * CPT + ph * CPH, CPH)], srcv)

            def descs(r, b):
                return (
                    pltpu.make_async_copy(xn_hbm.at[tgtv.at[r]], ft[b], sf[b]),
                    pltpu.make_async_copy(xn_hbm.at[srcv.at[r]], fs[b], ss[b]))

            def issue(r, b):
                d1, d2 = descs(r, b)
                d1.start()
                d2.start()

            def compute_tail(r, b):
                d1, d2 = descs(r, b)
                d1.wait()
                d2.wait()

                @plsc.parallel_loop(0, CHUNK, unroll=2)
                def _(i):
                    lane = lax.iota(jnp.int32, HP)
                    row = jnp.zeros((HP,), jnp.float32)
                    for h in range(H):
                        z = (ft[b][i, pl.ds(h * U, U)]
                             + fs[b][i, pl.ds(h * U, U)]) + tb[h]
                        t = jnp.maximum(z, 0.2 * z) * ka[h]
                        row = jnp.where(lane == h, jnp.sum(t), row)
                    p16 = jnp.where(lane < H, jnp.exp(row), 0.0)
                    p_v[i, :] = p16
                    for h in range(H):
                        fs[b][i, pl.ds(h * U, U)] = (
                            fs[b][i, pl.ds(h * U, U)] * p16[h])

                pltpu.sync_copy(p_v, acc16.at[tgtv.at[r]], add=True)
                pltpu.sync_copy(fs[b], acc128.at[tgtv.at[r]], add=True)

            issue(0, 0)

            @pl.loop(0, CPH, step=2)
            def _(rr):
                issue(rr + 1, 1)
                compute_tail(rr, 0)

                @pl.when(rr + 2 < CPH)
                def _():
                    issue(rr + 2, 0)

                compute_tail(rr + 1, 1)

        plsc.subcore_barrier()
        pltpu.sync_copy(
            acc16.at[pl.ds(s * RPS, RPS)],
            ssum_hbm.at[c, pl.ds(s * RPS, RPS)])
        pltpu.sync_copy(
            acc128.at[pl.ds(s * RPS, RPS)],
            out_hbm.at[c, pl.ds(s * RPS, RPS)])

    return k(xn, tgt2d, src2d, ka1, tba, zeros16, zeros128)


def _tc_finish(ssum_p, acc, bias):
    rb = 1000

    def body(s_ref, a_ref, b_ref, o_ref):
        rinv = 1.0 / (s_ref[0, :, :H] + s_ref[1, :, :H] + 1e-7)
        rinvx = jnp.repeat(rinv, U, axis=1)
        o_ref[...] = jax.nn.gelu(
            (a_ref[0] + a_ref[1]) * rinvx + b_ref[...])

    return pl.pallas_call(
        body,
        grid=(N // rb,),
        in_specs=[
            pl.BlockSpec((NC, rb, HP), lambda i: (0, i, 0)),
            pl.BlockSpec((NC, rb, D), lambda i: (0, i, 0)),
            pl.BlockSpec((1, D), lambda i: (0, 0)),
        ],
        out_specs=pl.BlockSpec((rb, D), lambda i: (i, 0)),
        out_shape=jax.ShapeDtypeStruct((N, D), jnp.float32),
    )(ssum_p, acc, bias.reshape(1, D))


def kernel(x, edges, kernel, kernel_attention1, bias_attention, bias):
    w = kernel.reshape(D, H * U)
    ka1 = kernel_attention1.reshape(H * U)
    tba = 2.0 * bias_attention.reshape(H * U)
    pad = N + (jnp.arange(E_PAD - E, dtype=jnp.int32) % (NP - N))
    tgt2d = jnp.concatenate([edges[:, 1], pad]).reshape(NCH, CHUNK)
    src2d = jnp.concatenate([edges[:, 0], pad]).reshape(NCH, CHUNK)
    xp = jnp.pad(x, ((0, NP - N), (0, 0)))
    zeros16 = jnp.zeros((NP, HP), jnp.float32)
    zeros128 = jnp.zeros((NP, D), jnp.float32)

    xn = _tc_project(xp, w)
    ssum_p, acc = _sc_fused(xn, tgt2d, src2d, ka1, tba, zeros16, zeros128)
    return _tc_finish(ssum_p, acc, bias)
